# Initial kernel scaffold; baseline (speedup 1.0000x reference)
import jax
import jax.numpy as jnp
from jax.experimental import pallas as pl


def kernel(input_labels, out_labels, noise_idx, num_sampled, node_table, ctx_table):
    raise NotImplementedError("write your pallas kernel here")


# speedup vs baseline: 1.1459x; 1.1459x over previous
import functools

import jax
import jax.numpy as jnp
from jax import lax
from jax.experimental import pallas as pl
from jax.experimental.pallas import tpu as pltpu
from jax.experimental.pallas import tpu_sc as plsc

D = 32
NS = 5
NC = 2
NSUB = 16
NW = NC * NSUB
CH = 256
GRP = 16


def _sc_logits(node_table, ctx_table, nid, oid, xid, r_total):
    rw = r_total // NW
    nchunk = rw // CH

    mesh = plsc.VectorSubcoreMesh(
        core_axis_name="c", subcore_axis_name="s",
        num_cores=NC, num_subcores=NSUB)

    @functools.partial(
        pl.kernel,
        out_type=jax.ShapeDtypeStruct((6, r_total), jnp.float32),
        mesh=mesh,
        compiler_params=pltpu.CompilerParams(
            needs_layout_passes=False, use_tc_tiling_on_sc=False),
        scratch_types=[
            pltpu.VMEM((rw,), jnp.int32),
            pltpu.VMEM((rw,), jnp.int32),
            pltpu.VMEM((rw * NS,), jnp.int32),
            pltpu.VMEM((CH, D), jnp.float32),
            pltpu.VMEM((CH, D), jnp.float32),
            pltpu.VMEM((CH * NS, D), jnp.float32),
            pltpu.VMEM((6 * rw,), jnp.float32),
            pltpu.SemaphoreType.DMA,
        ],
    )
    def body(node_hbm, ctx_hbm, nid_hbm, oid_hbm, xid_hbm, t_hbm,
             nidx_v, oidx_v, xidx_v, node_v, out_v, noise_v, t_v, sem):
        wid = lax.axis_index("s") * NC + lax.axis_index("c")
        pltpu.sync_copy(nid_hbm.at[pl.ds(wid * rw, rw)], nidx_v)
        pltpu.sync_copy(oid_hbm.at[pl.ds(wid * rw, rw)], oidx_v)
        pltpu.sync_copy(xid_hbm.at[pl.ds(wid * rw * NS, rw * NS)], xidx_v)

        lane = lax.iota(jnp.int32, GRP)

        def chunk_body(c, carry):
            cps = []
            for j in range(CH // 128):
                cps.append(pltpu.async_copy(
                    node_hbm.at[nidx_v.at[pl.ds(c * CH + j * 128, 128)]],
                    node_v.at[pl.ds(j * 128, 128)], sem))
                cps.append(pltpu.async_copy(
                    ctx_hbm.at[oidx_v.at[pl.ds(c * CH + j * 128, 128)]],
                    out_v.at[pl.ds(j * 128, 128)], sem))
            for j in range(CH * NS // 128):
                cps.append(pltpu.async_copy(
                    ctx_hbm.at[xidx_v.at[pl.ds(c * CH * NS + j * 128, 128)]],
                    noise_v.at[pl.ds(j * 128, 128)], sem))
            for cp in cps:
                cp.wait()

            def group_body(g, gcarry):
                row16 = g * GRP + lane
                nrows = [row16 * NS + s for s in range(NS)]
                accs = [jnp.zeros((GRP,), jnp.float32) for _ in range(6)]
                for d in range(D):
                    dcol = jnp.full((GRP,), d, jnp.int32)
                    vi = plsc.load_gather(node_v, [row16, dcol])
                    vo = plsc.load_gather(out_v, [row16, dcol])
                    accs[0] = accs[0] + vi * vo
                    for s in range(NS):
                        vn = plsc.load_gather(noise_v, [nrows[s], dcol])
                        accs[1 + s] = accs[1 + s] + vi * vn
                base = c * CH + g * GRP
                for k in range(6):
                    t_v[pl.ds(k * rw + base, GRP)] = accs[k]
                return gcarry

            lax.fori_loop(0, CH // GRP, group_body, 0)
            return carry

        lax.fori_loop(0, nchunk, chunk_body, 0)
        for k in range(6):
            pltpu.sync_copy(t_v.at[pl.ds(k * rw, rw)],
                            t_hbm.at[k, pl.ds(wid * rw, rw)])

    return body(node_table, ctx_table, nid, oid, xid)


def _tc_reduce(t, batch):
    def body(t_ref, o_ref):
        x = t_ref[...]
        pos = x[0:1, :]
        neg = x[1:6, :]

        def logsig(z):
            return jnp.minimum(z, 0.0) - jnp.log(1.0 + jnp.exp(-jnp.abs(z)))

        total = jnp.sum(logsig(pos)) + jnp.sum(logsig(-neg))
        o_ref[0, 0] = -total / batch

    out = pl.pallas_call(
        body,
        out_shape=jax.ShapeDtypeStruct((1, 1), jnp.float32),
        out_specs=pl.BlockSpec(memory_space=pltpu.SMEM),
    )(t)
    return out[0, 0]


def kernel(input_labels, out_labels, noise_idx, num_sampled, node_table,
           ctx_table):
    b, w = out_labels.shape
    r_total = b * w
    nid = jnp.tile(input_labels.astype(jnp.int32), w)
    oid = out_labels.reshape(-1).astype(jnp.int32)
    xid = noise_idx.astype(jnp.int32).reshape(-1)
    t = _sc_logits(node_table, ctx_table, nid, oid, xid, r_total)
    return _tc_reduce(t, b)
